# baseline (speedup 1.0000x reference)
import functools

import jax
import jax.numpy as jnp
import numpy as np
from jax import lax
from jax.experimental import pallas as pl
from jax.experimental.pallas import tpu as pltpu
from jax.experimental.pallas import tpu_sc as plsc

NC = 2
NS = 16
NW = NC * NS
SEQ_PAD = 64
SEQ_CHUNK = 16


def _gather_body(n_seq, seq_len, emb, x_hbm, table_hbm, out_hbm,
                 idx_v, rows_v, sem, sem2):
    wid = lax.axis_index("s") * NC + lax.axis_index("c")
    seq_per_w = n_seq // NW
    n_chunks = seq_per_w // SEQ_CHUNK
    seq0 = wid * seq_per_w

    def chunk(i, carry):
        s = seq0 + i * SEQ_CHUNK
        pltpu.sync_copy(x_hbm.at[pl.ds(s, SEQ_CHUNK)], idx_v)
        gathers = [
            pltpu.async_copy(table_hbm.at[idx_v.at[q]], rows_v.at[q], sem)
            for q in range(SEQ_CHUNK)
        ]
        for d in gathers:
            d.wait()
        writes = [
            pltpu.async_copy(rows_v.at[q, pl.ds(0, 56)],
                             out_hbm.at[s + q], sem2)
            for q in range(SEQ_CHUNK)
        ]
        for d in writes:
            d.wait()
        return carry

    lax.fori_loop(0, n_chunks, chunk, 0)


def kernel(x, table):
    n_seq, seq_len = x.shape
    vocab, emb = table.shape

    sel = np.zeros((seq_len, SEQ_PAD), np.float32)
    for c in range(SEQ_PAD):
        sel[c if c < seq_len else c - seq_len, c] = 1.0
    xp = lax.dot(x.astype(jnp.float32), jnp.asarray(sel),
                 precision=lax.Precision.HIGHEST).astype(jnp.int32)

    one = lax.optimization_barrier(jnp.ones((), jnp.float32))
    table = table * one

    embed = pl.kernel(
        functools.partial(_gather_body, n_seq, seq_len, emb),
        out_type=jax.ShapeDtypeStruct((n_seq, 56, emb), jnp.float32),
        mesh=plsc.VectorSubcoreMesh(core_axis_name="c", subcore_axis_name="s"),
        compiler_params=pltpu.CompilerParams(use_tc_tiling_on_sc=False),
        scratch_types=[
            pltpu.VMEM((SEQ_CHUNK, SEQ_PAD), jnp.int32),
            pltpu.VMEM((SEQ_CHUNK, SEQ_PAD, emb), jnp.float32),
            pltpu.SemaphoreType.DMA,
            pltpu.SemaphoreType.DMA,
        ],
    )
    return embed(xp, table)[:, :seq_len, :] * one


# speedup vs baseline: 1.4968x; 1.4968x over previous
import functools

import jax
import jax.numpy as jnp
import numpy as np
from jax import lax
from jax.experimental import pallas as pl
from jax.experimental.pallas import tpu as pltpu
from jax.experimental.pallas import tpu_sc as plsc

NC = 2
NS = 16
NW = NC * NS
SEQ_PAD = 64
SEQ_CHUNK = 16


def _gather_body(n_seq, seq_len, emb, x_hbm, table_hbm, out_hbm,
                 idx_v, rows_v, sem, sem2):
    wid = lax.axis_index("s") * NC + lax.axis_index("c")
    seq_per_w = n_seq // NW
    n_chunks = seq_per_w // SEQ_CHUNK
    seq0 = wid * seq_per_w

    def chunk(i, carry):
        s = seq0 + i * SEQ_CHUNK
        pltpu.sync_copy(x_hbm.at[pl.ds(s, SEQ_CHUNK)], idx_v)
        gathers = [
            pltpu.async_copy(table_hbm.at[idx_v.at[q]], rows_v.at[q], sem)
            for q in range(SEQ_CHUNK)
        ]
        for d in gathers:
            d.wait()
        writes = [
            pltpu.async_copy(rows_v.at[q, pl.ds(0, 56)],
                             out_hbm.at[s + q], sem2)
            for q in range(SEQ_CHUNK)
        ]
        for d in writes:
            d.wait()
        return carry

    lax.fori_loop(0, n_chunks, chunk, 0)


def kernel(x, table):
    n_seq, seq_len = x.shape
    vocab, emb = table.shape

    sel = np.zeros((seq_len, SEQ_PAD), np.float32)
    for c in range(SEQ_PAD):
        sel[c if c < seq_len else c - seq_len, c] = 1.0
    xp = lax.dot(x.astype(jnp.float32), jnp.asarray(sel),
                 precision=lax.Precision.HIGHEST).astype(jnp.int32)

    table = lax.dot(table, jnp.eye(emb, dtype=table.dtype))

    embed = pl.kernel(
        functools.partial(_gather_body, n_seq, seq_len, emb),
        out_type=jax.ShapeDtypeStruct((n_seq, 56, emb), jnp.float32),
        mesh=plsc.VectorSubcoreMesh(core_axis_name="c", subcore_axis_name="s"),
        compiler_params=pltpu.CompilerParams(use_tc_tiling_on_sc=False),
        scratch_types=[
            pltpu.VMEM((SEQ_CHUNK, SEQ_PAD), jnp.int32),
            pltpu.VMEM((SEQ_CHUNK, SEQ_PAD, emb), jnp.float32),
            pltpu.SemaphoreType.DMA,
            pltpu.SemaphoreType.DMA,
        ],
    )
    return embed(xp, table)[:, :seq_len, :]


# speedup vs baseline: 1.6043x; 1.0718x over previous
import functools

import jax
import jax.numpy as jnp
import numpy as np
from jax import lax
from jax.experimental import pallas as pl
from jax.experimental.pallas import tpu as pltpu
from jax.experimental.pallas import tpu_sc as plsc

NC = 2
NS = 16
NW = NC * NS
SEQ_PAD = 64
SEQ_CHUNK = 16
NBUF = 2


def _gather_body(n_seq, seq_len, emb, x_hbm, table_hbm, out_hbm,
                 idx_v, rows_v, gsem, wsems):
    wid = lax.axis_index("s") * NC + lax.axis_index("c")
    seq_per_w = n_seq // NW
    n_chunks = seq_per_w // SEQ_CHUNK
    seq0 = wid * seq_per_w

    def do_chunk(i, b, drain_writes):
        s = seq0 + i * SEQ_CHUNK
        pltpu.sync_copy(x_hbm.at[pl.ds(s, SEQ_CHUNK)], idx_v.at[b])
        if drain_writes:
            for q in range(SEQ_CHUNK):
                pltpu.make_async_copy(
                    rows_v.at[b, q, pl.ds(0, seq_len)],
                    out_hbm.at[s + q], wsems[b]).wait()
        gathers = [
            pltpu.async_copy(table_hbm.at[idx_v.at[b, q]],
                             rows_v.at[b, q], gsem)
            for q in range(SEQ_CHUNK)
        ]
        for d in gathers:
            d.wait()
        for q in range(SEQ_CHUNK):
            pltpu.async_copy(rows_v.at[b, q, pl.ds(0, seq_len)],
                             out_hbm.at[s + q], wsems[b])

    def pair(g, carry):
        do_chunk(2 * g, 0, True)
        do_chunk(2 * g + 1, 1, True)
        return carry

    do_chunk(0, 0, False)
    do_chunk(1, 1, False)
    lax.fori_loop(1, n_chunks // 2, pair, 0)
    for b in range(NBUF):
        for q in range(SEQ_CHUNK):
            pltpu.make_async_copy(
                rows_v.at[b, q, pl.ds(0, seq_len)],
                out_hbm.at[seq0 + q], wsems[b]).wait()


def kernel(x, table):
    n_seq, seq_len = x.shape
    vocab, emb = table.shape

    sel = np.zeros((seq_len, SEQ_PAD), np.float32)
    for c in range(SEQ_PAD):
        sel[c if c < seq_len else c - seq_len, c] = 1.0
    xp = lax.dot(x.astype(jnp.float32), jnp.asarray(sel),
                 precision=lax.Precision.HIGHEST).astype(jnp.int32)

    embed = pl.kernel(
        functools.partial(_gather_body, n_seq, seq_len, emb),
        out_type=jax.ShapeDtypeStruct((n_seq, seq_len, emb), jnp.float32),
        mesh=plsc.VectorSubcoreMesh(core_axis_name="c", subcore_axis_name="s"),
        compiler_params=pltpu.CompilerParams(use_tc_tiling_on_sc=False),
        scratch_types=[
            pltpu.VMEM((NBUF, SEQ_CHUNK, SEQ_PAD), jnp.int32),
            pltpu.VMEM((NBUF, SEQ_CHUNK, SEQ_PAD, emb), jnp.float32),
            pltpu.SemaphoreType.DMA,
            [pltpu.SemaphoreType.DMA, pltpu.SemaphoreType.DMA],
        ],
    )
    return embed(xp, table)


# speedup vs baseline: 1.6115x; 1.0045x over previous
import functools

import jax
import jax.numpy as jnp
import numpy as np
from jax import lax
from jax.experimental import pallas as pl
from jax.experimental.pallas import tpu as pltpu
from jax.experimental.pallas import tpu_sc as plsc

NC = 2
NS = 16
NW = NC * NS
SEQ_PAD = 56
SEQ_CHUNK = 16
NBUF = 2


def _gather_body(n_seq, seq_len, emb, x_hbm, table_hbm, out_hbm,
                 idx_v, rows_v, gsem, wsems):
    wid = lax.axis_index("s") * NC + lax.axis_index("c")
    seq_per_w = n_seq // NW
    n_chunks = seq_per_w // SEQ_CHUNK
    seq0 = wid * seq_per_w

    def do_chunk(i, b, drain_writes):
        s = seq0 + i * SEQ_CHUNK
        pltpu.sync_copy(x_hbm.at[pl.ds(s, SEQ_CHUNK)], idx_v.at[b])
        if drain_writes:
            for q in range(SEQ_CHUNK):
                pltpu.make_async_copy(
                    rows_v.at[b, q, pl.ds(0, seq_len)],
                    out_hbm.at[s + q], wsems[b]).wait()
        gathers = [
            pltpu.async_copy(table_hbm.at[idx_v.at[b, q]],
                             rows_v.at[b, q], gsem)
            for q in range(SEQ_CHUNK)
        ]
        for d in gathers:
            d.wait()
        for q in range(SEQ_CHUNK):
            pltpu.async_copy(rows_v.at[b, q, pl.ds(0, seq_len)],
                             out_hbm.at[s + q], wsems[b])

    def pair(g, carry):
        do_chunk(2 * g, 0, True)
        do_chunk(2 * g + 1, 1, True)
        return carry

    do_chunk(0, 0, False)
    do_chunk(1, 1, False)
    lax.fori_loop(1, n_chunks // 2, pair, 0)
    for b in range(NBUF):
        for q in range(SEQ_CHUNK):
            pltpu.make_async_copy(
                rows_v.at[b, q, pl.ds(0, seq_len)],
                out_hbm.at[seq0 + q], wsems[b]).wait()


def kernel(x, table):
    n_seq, seq_len = x.shape
    vocab, emb = table.shape

    sel = np.zeros((seq_len, SEQ_PAD), np.float32)
    for c in range(SEQ_PAD):
        sel[c if c < seq_len else c - seq_len, c] = 1.0
    xp = lax.dot(x.astype(jnp.float32), jnp.asarray(sel),
                 precision=lax.Precision.HIGHEST).astype(jnp.int32)

    embed = pl.kernel(
        functools.partial(_gather_body, n_seq, seq_len, emb),
        out_type=jax.ShapeDtypeStruct((n_seq, seq_len, emb), jnp.float32),
        mesh=plsc.VectorSubcoreMesh(core_axis_name="c", subcore_axis_name="s"),
        compiler_params=pltpu.CompilerParams(use_tc_tiling_on_sc=False),
        scratch_types=[
            pltpu.VMEM((NBUF, SEQ_CHUNK, SEQ_PAD), jnp.int32),
            pltpu.VMEM((NBUF, SEQ_CHUNK, SEQ_PAD, emb), jnp.float32),
            pltpu.SemaphoreType.DMA,
            [pltpu.SemaphoreType.DMA, pltpu.SemaphoreType.DMA],
        ],
    )
    return embed(xp, table)
